# baseline (speedup 1.0000x reference)
import functools

import jax
import jax.numpy as jnp
from jax import lax
from jax.experimental import pallas as pl
from jax.experimental.pallas import tpu as pltpu
from jax.experimental.pallas import tpu_sc as plsc

_B, _S, _D, _R = 4096, 50, 64, 16
_V = 1000000
_N = _B * _S
_NW = 32
_BPW = _B // _NW
_NCH = _BPW // 2
_CH = 2 * _S
_L = 16
_PACK_BLK = 8192


def _tc_pack(table, lora_A):
    def body(t_ref, a_ref, o_ref):
        z = jnp.zeros((_PACK_BLK, 128 - _D - _R), jnp.float32)
        o_ref[...] = jnp.concatenate([t_ref[...], a_ref[...], z], axis=1)

    return pl.pallas_call(
        body,
        grid=(_V // _PACK_BLK,),
        in_specs=[
            pl.BlockSpec((_PACK_BLK, _D), lambda i: (i, 0)),
            pl.BlockSpec((_PACK_BLK, _R), lambda i: (i, 0)),
        ],
        out_specs=pl.BlockSpec((_PACK_BLK, 128), lambda i: (i, 0)),
        out_shape=jax.ShapeDtypeStruct((_V, 128), jnp.float32),
    )(table, lora_A)


def _sc_embed_lora(x_flat, packed, lora_B):
    mesh = plsc.VectorSubcoreMesh(core_axis_name="c", subcore_axis_name="s")

    @functools.partial(
        pl.kernel,
        out_type=jax.ShapeDtypeStruct((_B, _S, _D), jnp.float32),
        mesh=mesh,
        compiler_params=pltpu.CompilerParams(needs_layout_passes=False),
        scratch_types=[
            pltpu.VMEM((_BPW * _S + 64,), jnp.int32),
            pltpu.VMEM((_NCH, 128), jnp.int32),
            pltpu.VMEM((128, 128), jnp.float32),
            pltpu.VMEM((128, 128), jnp.float32),
            pltpu.VMEM((104, _D), jnp.float32),
            pltpu.VMEM((104, _D), jnp.float32),
            pltpu.VMEM((_R, _D), jnp.float32),
            pltpu.SemaphoreType.DMA,
            pltpu.SemaphoreType.DMA,
            pltpu.SemaphoreType.DMA,
            pltpu.SemaphoreType.DMA,
        ],
    )
    def k(x_hbm, pk_hbm, b_hbm, out_hbm,
          idx_v, ixr2, p0, p1, o0, o1, b_v, sg0, sg1, so0, so1):
        wid = lax.axis_index("s") * 2 + lax.axis_index("c")
        batch0 = wid * _BPW
        pltpu.sync_copy(b_hbm, b_v)
        pltpu.sync_copy(x_hbm.at[pl.ds(batch0 * _S, _BPW * _S)],
                        idx_v.at[pl.ds(0, _BPW * _S)])
        for kk in range(4):
            idx_v[pl.ds(_BPW * _S + kk * _L, _L)] = jnp.zeros((_L,), jnp.int32)

        def fill_body(g2, cc):
            for kk in range(8):
                ids = lax.iota(jnp.int32, _L) + (g2 * _CH + kk * _L)
                v = plsc.load_gather(idx_v, [ids])
                ixr2[g2, pl.ds(kk * _L, _L)] = v
            return cc

        lax.fori_loop(0, _NCH, fill_body, 0)

        pbuf = (p0, p1)
        obuf = (o0, o1)
        sgs = (sg0, sg1)
        sos = (so0, so1)

        def issue_gather(g, c):
            pltpu.async_copy(pk_hbm.at[ixr2.at[g]], pbuf[c], sgs[c])

        def wait_gather(g, c):
            pltpu.make_async_copy(pk_hbm.at[ixr2.at[g]], pbuf[c],
                                  sgs[c]).wait()

        def issue_store(g, c):
            for h in range(2):
                pltpu.async_copy(obuf[c].at[pl.ds(h * _S, _S)],
                                 out_hbm.at[batch0 + 2 * g + h], sos[c])

        def wait_store(c):
            for h in range(2):
                pltpu.make_async_copy(obuf[c].at[pl.ds(h * _S, _S)],
                                      out_hbm.at[batch0 + h], sos[c]).wait()

        def compute_chunk(c):
            for p in range(2):
                bv = [(b_v[r, pl.ds(32 * p, 16)],
                       b_v[r, pl.ds(32 * p + 16, 16)]) for r in range(_R)]

                def row_body(i4, cc, bv=bv, p=p):
                    for u in range(4):
                        i = i4 * 4 + u
                        a_vec = pbuf[c][i, pl.ds(_D, 16)]
                        acc0 = pbuf[c][i, pl.ds(32 * p, 16)]
                        acc1 = pbuf[c][i, pl.ds(32 * p + 16, 16)]
                        for r in range(_R):
                            s = a_vec[r]
                            acc0 = acc0 + s * bv[r][0]
                            acc1 = acc1 + s * bv[r][1]
                        obuf[c][i, pl.ds(32 * p, 16)] = acc0
                        obuf[c][i, pl.ds(32 * p + 16, 16)] = acc1
                    return cc

                lax.fori_loop(0, _CH // 4, row_body, 0)

        issue_gather(0, 0)
        issue_gather(1, 1)

        def body(t, carry):
            for c in range(2):
                g = 2 * t + c
                wait_gather(g, c)

                @pl.when(t > 0)
                def _():
                    wait_store(c)

                compute_chunk(c)
                issue_store(g, c)

                @pl.when(g + 2 < _NCH)
                def _():
                    issue_gather(g + 2, c)
            return carry

        lax.fori_loop(0, _NCH // 2, body, 0)
        wait_store(0)
        wait_store(1)

    return k(x_flat, packed, lora_B)


def kernel(x, table, lora_A, lora_B):
    x_flat = x.reshape(-1).astype(jnp.int32)
    zeros = jnp.zeros((_V, 128 - _D - _R), jnp.float32)
    packed = jnp.concatenate([table, lora_A, zeros], axis=1)
    return _sc_embed_lora(x_flat, packed, lora_B)


# speedup vs baseline: 1.0062x; 1.0062x over previous
import functools

import jax
import jax.numpy as jnp
from jax import lax
from jax.experimental import pallas as pl
from jax.experimental.pallas import tpu as pltpu
from jax.experimental.pallas import tpu_sc as plsc

_B, _S, _D, _R = 4096, 50, 64, 16
_V = 1000000
_N = _B * _S
_NW = 32
_BPW = _B // _NW
_NCH = _BPW // 2
_CH = 2 * _S
_L = 16
_PACK_BLK = 8192


def _tc_pack(table, lora_A):
    def body(t_ref, a_ref, o_ref):
        z = jnp.zeros((_PACK_BLK, 128 - _D - _R), jnp.float32)
        o_ref[...] = jnp.concatenate([t_ref[...], a_ref[...], z], axis=1)

    return pl.pallas_call(
        body,
        grid=(_V // _PACK_BLK,),
        in_specs=[
            pl.BlockSpec((_PACK_BLK, _D), lambda i: (i, 0)),
            pl.BlockSpec((_PACK_BLK, _R), lambda i: (i, 0)),
        ],
        out_specs=pl.BlockSpec((_PACK_BLK, 128), lambda i: (i, 0)),
        out_shape=jax.ShapeDtypeStruct((_V, 128), jnp.float32),
    )(table, lora_A)


def _sc_embed_lora(x_flat, packed, lora_B):
    mesh = plsc.VectorSubcoreMesh(core_axis_name="c", subcore_axis_name="s")

    @functools.partial(
        pl.kernel,
        out_type=jax.ShapeDtypeStruct((_B, _S, _D), jnp.float32),
        mesh=mesh,
        compiler_params=pltpu.CompilerParams(needs_layout_passes=False),
        scratch_types=[
            pltpu.VMEM((_BPW * _S + 64,), jnp.int32),
            pltpu.VMEM((_NCH, 128), jnp.int32),
            pltpu.VMEM((128, 128), jnp.float32),
            pltpu.VMEM((128, 128), jnp.float32),
            pltpu.VMEM((128, 128), jnp.float32),
            pltpu.VMEM((128, 128), jnp.float32),
            pltpu.VMEM((104, _D), jnp.float32),
            pltpu.VMEM((104, _D), jnp.float32),
            pltpu.VMEM((_R, _D), jnp.float32),
            pltpu.SemaphoreType.DMA,
            pltpu.SemaphoreType.DMA,
            pltpu.SemaphoreType.DMA,
            pltpu.SemaphoreType.DMA,
            pltpu.SemaphoreType.DMA,
            pltpu.SemaphoreType.DMA,
        ],
    )
    def k(x_hbm, pk_hbm, b_hbm, out_hbm,
          idx_v, ixr2, p0, p1, p2, p3, o0, o1, b_v,
          sg0, sg1, sg2, sg3, so0, so1):
        wid = lax.axis_index("s") * 2 + lax.axis_index("c")
        batch0 = wid * _BPW
        pltpu.sync_copy(b_hbm, b_v)
        pltpu.sync_copy(x_hbm.at[pl.ds(batch0 * _S, _BPW * _S)],
                        idx_v.at[pl.ds(0, _BPW * _S)])
        for kk in range(4):
            idx_v[pl.ds(_BPW * _S + kk * _L, _L)] = jnp.zeros((_L,), jnp.int32)

        def fill_body(g2, cc):
            for kk in range(8):
                ids = lax.iota(jnp.int32, _L) + (g2 * _CH + kk * _L)
                v = plsc.load_gather(idx_v, [ids])
                ixr2[g2, pl.ds(kk * _L, _L)] = v
            return cc

        lax.fori_loop(0, _NCH, fill_body, 0)

        pbuf = (p0, p1, p2, p3)
        obuf = (o0, o1)
        sgs = (sg0, sg1, sg2, sg3)
        sos = (so0, so1)

        def issue_gather(g, c):
            pltpu.async_copy(pk_hbm.at[ixr2.at[g]], pbuf[c], sgs[c])

        def wait_gather(g, c):
            pltpu.make_async_copy(pk_hbm.at[ixr2.at[g]], pbuf[c],
                                  sgs[c]).wait()

        def issue_store(g, c):
            for h in range(2):
                pltpu.async_copy(obuf[c].at[pl.ds(h * _S, _S)],
                                 out_hbm.at[batch0 + 2 * g + h], sos[c])

        def wait_store(c):
            for h in range(2):
                pltpu.make_async_copy(obuf[c].at[pl.ds(h * _S, _S)],
                                      out_hbm.at[batch0 + h], sos[c]).wait()

        def compute_chunk(c, co):
            for p in range(2):
                bv = [(b_v[r, pl.ds(32 * p, 16)],
                       b_v[r, pl.ds(32 * p + 16, 16)]) for r in range(_R)]

                def row_body(i, cc, bv=bv, p=p, co=co):
                    a_vec = pbuf[c][i, pl.ds(_D, 16)]
                    acc0 = pbuf[c][i, pl.ds(32 * p, 16)]
                    acc1 = pbuf[c][i, pl.ds(32 * p + 16, 16)]
                    for r in range(_R):
                        s = a_vec[r]
                        acc0 = acc0 + s * bv[r][0]
                        acc1 = acc1 + s * bv[r][1]
                    obuf[co][i, pl.ds(32 * p, 16)] = acc0
                    obuf[co][i, pl.ds(32 * p + 16, 16)] = acc1
                    return cc

                lax.fori_loop(0, _CH, row_body, 0)

        for u in range(4):
            issue_gather(u, u)

        def body(t, carry):
            for u in range(4):
                g = 4 * t + u
                wait_gather(g, u)

                @pl.when(g >= 2)
                def _():
                    wait_store(u % 2)

                compute_chunk(u, u % 2)
                issue_store(g, u % 2)

                @pl.when(g + 4 < _NCH)
                def _():
                    issue_gather(g + 4, u)
            return carry

        lax.fori_loop(0, _NCH // 4, body, 0)
        wait_store(0)
        wait_store(1)

    return k(x_flat, packed, lora_B)


def kernel(x, table, lora_A, lora_B):
    x_flat = x.reshape(-1).astype(jnp.int32)
    zeros = jnp.zeros((_V, 128 - _D - _R), jnp.float32)
    packed = jnp.concatenate([table, lora_A, zeros], axis=1)
    return _sc_embed_lora(x_flat, packed, lora_B)


# speedup vs baseline: 1.0063x; 1.0001x over previous
import functools

import jax
import jax.numpy as jnp
from jax import lax
from jax.experimental import pallas as pl
from jax.experimental.pallas import tpu as pltpu
from jax.experimental.pallas import tpu_sc as plsc

_B, _S, _D, _R = 4096, 50, 64, 16
_V = 1000000
_N = _B * _S
_NW = 32
_BPW = _B // _NW
_NCH = _BPW // 2
_CH = 2 * _S
_L = 16


def _sc_embed_lora(x_flat, packed, lora_B):
    mesh = plsc.VectorSubcoreMesh(core_axis_name="c", subcore_axis_name="s")

    @functools.partial(
        pl.kernel,
        out_type=jax.ShapeDtypeStruct((_B, _S, _D), jnp.float32),
        mesh=mesh,
        compiler_params=pltpu.CompilerParams(needs_layout_passes=False),
        scratch_types=[
            pltpu.VMEM((_BPW * _S + 64,), jnp.int32),
            pltpu.VMEM((_NCH, 128), jnp.int32),
            pltpu.VMEM((128, 128), jnp.float32),
            pltpu.VMEM((128, 128), jnp.float32),
            pltpu.VMEM((128, 128), jnp.float32),
            pltpu.VMEM((128, 128), jnp.float32),
            pltpu.VMEM((104, _D), jnp.float32),
            pltpu.VMEM((104, _D), jnp.float32),
            pltpu.VMEM((_R, _D), jnp.float32),
            pltpu.SemaphoreType.DMA,
            pltpu.SemaphoreType.DMA,
            pltpu.SemaphoreType.DMA,
            pltpu.SemaphoreType.DMA,
            pltpu.SemaphoreType.DMA,
            pltpu.SemaphoreType.DMA,
        ],
    )
    def k(x_hbm, pk_hbm, b_hbm, out_hbm,
          idx_v, ixr2, p0, p1, p2, p3, o0, o1, b_v,
          sg0, sg1, sg2, sg3, so0, so1):
        wid = lax.axis_index("s") * 2 + lax.axis_index("c")
        batch0 = wid * _BPW
        pltpu.sync_copy(b_hbm, b_v)
        pltpu.sync_copy(x_hbm.at[pl.ds(batch0 * _S, _BPW * _S)],
                        idx_v.at[pl.ds(0, _BPW * _S)])
        for kk in range(4):
            idx_v[pl.ds(_BPW * _S + kk * _L, _L)] = jnp.zeros((_L,), jnp.int32)

        def fill_body(g2, cc):
            for kk in range(8):
                ids = lax.iota(jnp.int32, _L) + (g2 * _CH + kk * _L)
                v = plsc.load_gather(idx_v, [ids])
                ixr2[g2, pl.ds(kk * _L, _L)] = v
            return cc

        lax.fori_loop(0, _NCH, fill_body, 0)

        pbuf = (p0, p1, p2, p3)
        obuf = (o0, o1)
        sgs = (sg0, sg1, sg2, sg3)
        sos = (so0, so1)

        def issue_gather(g, c):
            pltpu.async_copy(pk_hbm.at[ixr2.at[g]], pbuf[c], sgs[c])

        def wait_gather(g, c):
            pltpu.make_async_copy(pk_hbm.at[ixr2.at[g]], pbuf[c],
                                  sgs[c]).wait()

        def issue_store(g, c):
            for h in range(2):
                pltpu.async_copy(obuf[c].at[pl.ds(h * _S, _S)],
                                 out_hbm.at[batch0 + 2 * g + h], sos[c])

        def wait_store(c):
            for h in range(2):
                pltpu.make_async_copy(obuf[c].at[pl.ds(h * _S, _S)],
                                      out_hbm.at[batch0 + h], sos[c]).wait()

        def compute_chunk(c, co):
            for p in range(2):
                bv = [(b_v[r, pl.ds(32 * p, 16)],
                       b_v[r, pl.ds(32 * p + 16, 16)]) for r in range(_R)]

                def row_body(i, cc, bv=bv, p=p, co=co):
                    a_vec = pbuf[c][i, pl.ds(_D, 16)]
                    acc0 = pbuf[c][i, pl.ds(32 * p, 16)]
                    acc1 = pbuf[c][i, pl.ds(32 * p + 16, 16)]
                    for r in range(_R):
                        s = a_vec[r]
                        acc0 = acc0 + s * bv[r][0]
                        acc1 = acc1 + s * bv[r][1]
                    obuf[co][i, pl.ds(32 * p, 16)] = acc0
                    obuf[co][i, pl.ds(32 * p + 16, 16)] = acc1
                    return cc

                lax.fori_loop(0, _CH, row_body, 0)

        for u in range(4):
            issue_gather(u, u)

        def body(t, carry):
            for u in range(4):
                g = 4 * t + u
                wait_gather(g, u)

                @pl.when(g >= 2)
                def _():
                    wait_store(u % 2)

                compute_chunk(u, u % 2)
                issue_store(g, u % 2)

                @pl.when(g + 4 < _NCH)
                def _():
                    issue_gather(g + 4, u)
            return carry

        lax.fori_loop(0, _NCH // 4, body, 0)
        wait_store(0)
        wait_store(1)

    return k(x_flat, packed, lora_B)


def kernel(x, table, lora_A, lora_B):
    x_flat = x.reshape(-1).astype(jnp.int32)
    zeros = jnp.zeros((_V, 128 - _D - _R), jnp.float32)
    packed = jnp.concatenate([table, lora_A, zeros], axis=1)
    return _sc_embed_lora(x_flat, packed, lora_B)
